# baseline (speedup 1.0000x reference)
import math
from fractions import Fraction
import functools

import numpy as np
import jax
import jax.numpy as jnp
from jax import lax
from jax.experimental import pallas as pl
from jax.experimental.pallas import tpu as pltpu
from jax.experimental.pallas import tpu_sc as plsc

N_NODES = 50000
N_EDGES = 1600000
IRREP_DIMS = (1, 3, 5)
MAX_SH_DEGREE = 2
TOTAL_DIM = 9


def _su2_cg(j1m1, j2m2, j3m3):
    j1, m1 = j1m1
    j2, m2 = j2m2
    j3, m3 = j3m3
    if m3 != m1 + m2:
        return 0.0
    vmin = int(max(-j1 + j2 + m3, -j1 + m1, 0))
    vmax = int(min(j2 + j3 + m1, j3 - j1 + j2, j3 + m3))

    def f(n):
        return math.factorial(round(n))

    C = ((2.0 * j3 + 1.0) * Fraction(
        f(j3 + j1 - j2) * f(j3 - j1 + j2) * f(j1 + j2 - j3) * f(j3 + m3) * f(j3 - m3),
        f(j1 + j2 + j3 + 1) * f(j1 - m1) * f(j1 + m1) * f(j2 - m2) * f(j2 + m2))) ** 0.5
    S = 0
    for v in range(vmin, vmax + 1):
        S += (-1) ** int(v + j2 + m2) * Fraction(
            f(j2 + j3 + m1 - v) * f(j1 - m1 + v),
            f(v) * f(j3 - j1 + j2 - v) * f(j3 + m3 - v) * f(v + j1 - j2 - m3))
    return float(C * S)


def _su2_clebsch_gordan(j1, j2, j3):
    mat = np.zeros((2 * j1 + 1, 2 * j2 + 1, 2 * j3 + 1), dtype=np.float64)
    for m1 in range(-j1, j1 + 1):
        for m2 in range(-j2, j2 + 1):
            if abs(m1 + m2) <= j3:
                mat[j1 + m1, j2 + m2, j3 + m1 + m2] = _su2_cg(
                    (j1, m1), (j2, m2), (j3, m1 + m2))
    return mat


def _change_basis_real_to_complex(l):
    q = np.zeros((2 * l + 1, 2 * l + 1), dtype=np.complex128)
    for m in range(-l, 0):
        q[l + m, l + abs(m)] = 1 / 2 ** 0.5
        q[l + m, l - abs(m)] = -1j / 2 ** 0.5
    q[l, l] = 1
    for m in range(1, l + 1):
        q[l + m, l + abs(m)] = (-1) ** m / 2 ** 0.5
        q[l + m, l - abs(m)] = 1j * (-1) ** m / 2 ** 0.5
    return (-1j) ** l * q


def _so3_clebsch_gordan(l1, l2, l3):
    Q1 = _change_basis_real_to_complex(l1)
    Q2 = _change_basis_real_to_complex(l2)
    Q3 = _change_basis_real_to_complex(l3)
    C = _su2_clebsch_gordan(l1, l2, l3).astype(np.complex128)
    C = np.einsum('ij,kl,mn,ikn->jlm', Q1, Q2, np.conj(Q3.T), C)
    return np.real(C).astype(np.float32)


def _build_cg_combos():
    ls = [(d - 1) // 2 for d in IRREP_DIMS]
    combos = []
    for e in range(MAX_SH_DEGREE + 1):
        for ni in range(len(IRREP_DIMS)):
            for oi in range(len(IRREP_DIMS)):
                nl, ol = ls[ni], ls[oi]
                if not (abs(e - nl) <= ol <= e + nl):
                    continue
                cg = _so3_clebsch_gordan(e, nl, ol)
                if np.abs(cg).sum() > 1e-10:
                    combos.append((e, ni, oi, cg))
    return combos


def _build_tables():
    combos = _build_cg_combos()
    offs = [0, 1, 4, 9]
    terms = []
    for idx, (e, ni, oi, cg) in enumerate(combos):
        I, J, K = cg.shape
        for i in range(I):
            for j in range(J):
                for k in range(K):
                    v = float(cg[i, j, k])
                    if abs(v) > 1e-8:
                        terms.append((offs[e] + i, offs[ni] + j, offs[oi] + k, idx, v))
    coef_ids = {}
    coef_rows = []
    groups = [{} for _ in range(TOTAL_DIM)]
    for (i, j, k, c, v) in terms:
        key = (c, round(abs(v), 10))
        if key not in coef_ids:
            coef_ids[key] = len(coef_rows)
            coef_rows.append(key)
        cid = coef_ids[key]
        groups[k].setdefault(cid, []).append((i, j, 1 if v > 0 else -1))
    combo_idx = np.asarray([c for (c, av) in coef_rows], dtype=np.int32)
    absval = np.asarray([av for (c, av) in coef_rows], dtype=np.float32)
    groups = [list(g.items()) for g in groups]
    return combo_idx, absval, groups


_COEF_COMBO, _COEF_ABS, _GROUPS = _build_tables()
_N_COEF = _COEF_COMBO.shape[0]
_N_COEF_PAD = ((_N_COEF + 7) // 8) * 8


def _emit_bilinear(shv, xv, coef_of):
    prod_cache = {}

    def prod(i, j):
        if (i, j) not in prod_cache:
            prod_cache[(i, j)] = shv[i] * xv[j]
        return prod_cache[(i, j)]

    accs = []
    for k in range(TOTAL_DIM):
        acc = None
        for cid, tlist in _GROUPS[k]:
            tl = sorted(tlist, key=lambda t: -t[2])
            neg = tl[0][2] < 0
            if neg:
                tl = [(i, j, -s) for (i, j, s) in tl]
            br = None
            for (i, j, s) in tl:
                p = prod(i, j)
                if br is None:
                    br = p
                elif s > 0:
                    br = br + p
                else:
                    br = br - p
            v = coef_of(cid) * br
            if acc is None:
                acc = (-v) if neg else v
            elif neg:
                acc = acc - v
            else:
                acc = acc + v
        accs.append(acc)
    return accs


_NC = 2
_NS = 16
_NW = _NC * _NS
_EPW = N_EDGES // _NW
_B = 400
_NB = _EPW // _B
_GRP = _B // 16
_CHUNK = 80
_NCH = _B // _CHUNK
_ROWS_PER_TILE = 3128
_N_PAD = _ROWS_PER_TILE * _NS


def _sc_body(node_hbm, src_hbm, dst_hbm, sh0_hbm, sh1_hbm, sh2_hbm,
             coef_hbm, zeros_hbm, out_hbm,
             src_v, dst_v, rows_v, sh0_v, sh1_v, sh2_v, msg_v, coef_v,
             table_sh, sem):
    cid = lax.axis_index("c")
    sid = lax.axis_index("s")
    wid = sid * _NC + cid
    wbase = wid * _EPW

    pltpu.sync_copy(zeros_hbm, table_sh.at[pl.ds(sid * _ROWS_PER_TILE,
                                                 _ROWS_PER_TILE)])
    pltpu.sync_copy(coef_hbm, coef_v)

    init_row = jnp.where(lax.iota(jnp.int32, 16) == 9,
                         jnp.float32(1.0), jnp.float32(0.0))

    def init_body(i, _):
        msg_v[i] = init_row
        return 0

    lax.fori_loop(0, _B, init_body, 0)

    plsc.subcore_barrier()

    lanes = lax.iota(jnp.int32, 16)

    def block_body(blk, _):
        base = wbase + blk * _B
        crow = (wbase // _CHUNK) + blk * _NCH
        pltpu.sync_copy(src_hbm.at[pl.ds(crow, _NCH)], src_v)
        pltpu.sync_copy(dst_hbm.at[pl.ds(crow, _NCH)], dst_v)
        pltpu.sync_copy(sh0_hbm.at[pl.ds(base, _B), :], sh0_v)
        pltpu.sync_copy(sh1_hbm.at[pl.ds(base, _B), :], sh1_v)
        pltpu.sync_copy(sh2_hbm.at[pl.ds(base, _B), :], sh2_v)
        descs = [pltpu.async_copy(node_hbm.at[src_v.at[j]],
                                  rows_v.at[pl.ds(j * _CHUNK, _CHUNK)], sem)
                 for j in range(_NCH)]
        for d in descs:
            d.wait()

        def group_body(g, _):
            r = g * 16
            eidx = lanes + r
            shv = [plsc.load_gather(sh0_v,
                                    [eidx, jnp.full((16,), 0, jnp.int32)])]
            for i in range(3):
                shv.append(plsc.load_gather(
                    sh1_v, [eidx, jnp.full((16,), i, jnp.int32)]))
            for i in range(5):
                shv.append(plsc.load_gather(
                    sh2_v, [eidx, jnp.full((16,), i, jnp.int32)]))
            xv = [plsc.load_gather(
                rows_v, [eidx, jnp.full((16,), j, jnp.int32)])
                for j in range(TOTAL_DIM)]
            accs = _emit_bilinear(shv, xv, lambda cid_: coef_v[cid_])
            for k in range(TOTAL_DIM):
                plsc.store_scatter(
                    msg_v, [eidx, jnp.full((16,), k, jnp.int32)], accs[k])
            return 0

        lax.fori_loop(0, _GRP, group_body, 0)
        for j in range(_NCH):
            pltpu.sync_copy(msg_v.at[pl.ds(j * _CHUNK, _CHUNK)],
                            table_sh.at[dst_v.at[j]], add=True)
        return 0

    lax.fori_loop(0, _NB, block_body, 0)

    plsc.subcore_barrier()
    row0 = sid * _ROWS_PER_TILE
    pltpu.sync_copy(table_sh.at[pl.ds(row0, _ROWS_PER_TILE)],
                    out_hbm.at[cid, pl.ds(row0, _ROWS_PER_TILE)])


@jax.jit
def _sc_messages(node_pad, src, dst, sh0, sh1, sh2, coef_tab, zeros):
    mesh = plsc.VectorSubcoreMesh(core_axis_name="c", subcore_axis_name="s")
    return pl.kernel(
        _sc_body,
        out_type=jax.ShapeDtypeStruct((_NC, _N_PAD, 16), jnp.float32),
        mesh=mesh,
        compiler_params=pltpu.CompilerParams(needs_layout_passes=False,
                                             use_tc_tiling_on_sc=False),
        scratch_types=[
            pltpu.VMEM((_NCH, _CHUNK), jnp.int32),
            pltpu.VMEM((_NCH, _CHUNK), jnp.int32),
            pltpu.VMEM((_B, 16), jnp.float32),
            pltpu.VMEM((_B, 1), jnp.float32),
            pltpu.VMEM((_B, 3), jnp.float32),
            pltpu.VMEM((_B, 5), jnp.float32),
            pltpu.VMEM((_B, 16), jnp.float32),
            pltpu.VMEM((_N_COEF_PAD, 16), jnp.float32),
            pltpu.VMEM_SHARED((_N_PAD, 16), jnp.float32),
            pltpu.SemaphoreType.DMA,
        ],
    )(node_pad, src, dst, sh0, sh1, sh2, coef_tab, zeros)


_FR = 2000


def _fin_body(p_ref, o_ref):
    a = p_ref[0] + p_ref[1]
    cnt = jnp.maximum(a[:, 9:10], 1.0)
    o_ref[...] = a[:, :TOTAL_DIM] / cnt


@jax.jit
def _finalize(partials):
    return pl.pallas_call(
        _fin_body,
        grid=(N_NODES // _FR,),
        in_specs=[pl.BlockSpec((_NC, _FR, 16), lambda i: (0, i, 0))],
        out_specs=pl.BlockSpec((_FR, TOTAL_DIM), lambda i: (i, 0)),
        out_shape=jax.ShapeDtypeStruct((N_NODES, TOTAL_DIM), jnp.float32),
    )(partials)


def kernel(node_irreps, edge_index, sh_edge_0, sh_edge_1, sh_edge_2, w):
    node_pad = jnp.pad(node_irreps, ((0, 0), (0, 16 - TOTAL_DIM)))
    src = edge_index[0].reshape(N_EDGES // _CHUNK, _CHUNK)
    dst = edge_index[1].reshape(N_EDGES // _CHUNK, _CHUNK)
    sh0 = sh_edge_0
    sh1 = sh_edge_1
    sh2 = sh_edge_2
    coefs = w[jnp.asarray(_COEF_COMBO)] * jnp.asarray(_COEF_ABS)
    coefs = jnp.pad(coefs, (0, _N_COEF_PAD - _N_COEF))
    coef_tab = jnp.broadcast_to(coefs[:, None], (_N_COEF_PAD, 16))
    zeros = jnp.zeros((_ROWS_PER_TILE, 16), jnp.float32)
    partials = _sc_messages(node_pad, src, dst, sh0, sh1, sh2,
                            jnp.asarray(coef_tab), zeros)
    return _finalize(partials)


# speedup vs baseline: 1.3580x; 1.3580x over previous
import math
from fractions import Fraction
import functools

import numpy as np
import jax
import jax.numpy as jnp
from jax import lax
from jax.experimental import pallas as pl
from jax.experimental.pallas import tpu as pltpu
from jax.experimental.pallas import tpu_sc as plsc

N_NODES = 50000
N_EDGES = 1600000
IRREP_DIMS = (1, 3, 5)
MAX_SH_DEGREE = 2
TOTAL_DIM = 9


def _su2_cg(j1m1, j2m2, j3m3):
    j1, m1 = j1m1
    j2, m2 = j2m2
    j3, m3 = j3m3
    if m3 != m1 + m2:
        return 0.0
    vmin = int(max(-j1 + j2 + m3, -j1 + m1, 0))
    vmax = int(min(j2 + j3 + m1, j3 - j1 + j2, j3 + m3))

    def f(n):
        return math.factorial(round(n))

    C = ((2.0 * j3 + 1.0) * Fraction(
        f(j3 + j1 - j2) * f(j3 - j1 + j2) * f(j1 + j2 - j3) * f(j3 + m3) * f(j3 - m3),
        f(j1 + j2 + j3 + 1) * f(j1 - m1) * f(j1 + m1) * f(j2 - m2) * f(j2 + m2))) ** 0.5
    S = 0
    for v in range(vmin, vmax + 1):
        S += (-1) ** int(v + j2 + m2) * Fraction(
            f(j2 + j3 + m1 - v) * f(j1 - m1 + v),
            f(v) * f(j3 - j1 + j2 - v) * f(j3 + m3 - v) * f(v + j1 - j2 - m3))
    return float(C * S)


def _su2_clebsch_gordan(j1, j2, j3):
    mat = np.zeros((2 * j1 + 1, 2 * j2 + 1, 2 * j3 + 1), dtype=np.float64)
    for m1 in range(-j1, j1 + 1):
        for m2 in range(-j2, j2 + 1):
            if abs(m1 + m2) <= j3:
                mat[j1 + m1, j2 + m2, j3 + m1 + m2] = _su2_cg(
                    (j1, m1), (j2, m2), (j3, m1 + m2))
    return mat


def _change_basis_real_to_complex(l):
    q = np.zeros((2 * l + 1, 2 * l + 1), dtype=np.complex128)
    for m in range(-l, 0):
        q[l + m, l + abs(m)] = 1 / 2 ** 0.5
        q[l + m, l - abs(m)] = -1j / 2 ** 0.5
    q[l, l] = 1
    for m in range(1, l + 1):
        q[l + m, l + abs(m)] = (-1) ** m / 2 ** 0.5
        q[l + m, l - abs(m)] = 1j * (-1) ** m / 2 ** 0.5
    return (-1j) ** l * q


def _so3_clebsch_gordan(l1, l2, l3):
    Q1 = _change_basis_real_to_complex(l1)
    Q2 = _change_basis_real_to_complex(l2)
    Q3 = _change_basis_real_to_complex(l3)
    C = _su2_clebsch_gordan(l1, l2, l3).astype(np.complex128)
    C = np.einsum('ij,kl,mn,ikn->jlm', Q1, Q2, np.conj(Q3.T), C)
    return np.real(C).astype(np.float32)


def _build_cg_combos():
    ls = [(d - 1) // 2 for d in IRREP_DIMS]
    combos = []
    for e in range(MAX_SH_DEGREE + 1):
        for ni in range(len(IRREP_DIMS)):
            for oi in range(len(IRREP_DIMS)):
                nl, ol = ls[ni], ls[oi]
                if not (abs(e - nl) <= ol <= e + nl):
                    continue
                cg = _so3_clebsch_gordan(e, nl, ol)
                if np.abs(cg).sum() > 1e-10:
                    combos.append((e, ni, oi, cg))
    return combos


def _build_tables():
    combos = _build_cg_combos()
    offs = [0, 1, 4, 9]
    terms = []
    for idx, (e, ni, oi, cg) in enumerate(combos):
        I, J, K = cg.shape
        for i in range(I):
            for j in range(J):
                for k in range(K):
                    v = float(cg[i, j, k])
                    if abs(v) > 1e-8:
                        terms.append((offs[e] + i, offs[ni] + j, offs[oi] + k, idx, v))
    coef_ids = {}
    coef_rows = []
    groups = [{} for _ in range(TOTAL_DIM)]
    for (i, j, k, c, v) in terms:
        key = (c, round(abs(v), 10))
        if key not in coef_ids:
            coef_ids[key] = len(coef_rows)
            coef_rows.append(key)
        cid = coef_ids[key]
        groups[k].setdefault(cid, []).append((i, j, 1 if v > 0 else -1))
    combo_idx = np.asarray([c for (c, av) in coef_rows], dtype=np.int32)
    absval = np.asarray([av for (c, av) in coef_rows], dtype=np.float32)
    groups = [list(g.items()) for g in groups]
    return combo_idx, absval, groups


_COEF_COMBO, _COEF_ABS, _GROUPS = _build_tables()
_N_COEF = _COEF_COMBO.shape[0]
_N_COEF_PAD = ((_N_COEF + 7) // 8) * 8


def _emit_bilinear(shv, xv, coef_of):
    prod_cache = {}

    def prod(i, j):
        if (i, j) not in prod_cache:
            prod_cache[(i, j)] = shv[i] * xv[j]
        return prod_cache[(i, j)]

    accs = []
    for k in range(TOTAL_DIM):
        acc = None
        for cid, tlist in _GROUPS[k]:
            tl = sorted(tlist, key=lambda t: -t[2])
            neg = tl[0][2] < 0
            if neg:
                tl = [(i, j, -s) for (i, j, s) in tl]
            br = None
            for (i, j, s) in tl:
                p = prod(i, j)
                if br is None:
                    br = p
                elif s > 0:
                    br = br + p
                else:
                    br = br - p
            v = coef_of(cid) * br
            if acc is None:
                acc = (-v) if neg else v
            elif neg:
                acc = acc - v
            else:
                acc = acc + v
        accs.append(acc)
    return accs


_NC = 2
_NS = 16
_NW = _NC * _NS
_EPW = N_EDGES // _NW
_B = 400
_NB = _EPW // _B
_GRP = _B // 16
_CHUNK = 80
_NCH = _B // _CHUNK
_ROWS_PER_TILE = 3128
_N_PAD = _ROWS_PER_TILE * _NS


def _sc_body(node_hbm, ei_hbm, sh0_hbm, sh1_hbm, sh2_hbm,
             coef_hbm, zeros_hbm, out_hbm,
             src_v, dst_v, rows_v, sh0_v, sh1_v, sh2_v, msg_v, coef_v,
             table_sh, sem):
    cid = lax.axis_index("c")
    sid = lax.axis_index("s")
    wid = sid * _NC + cid
    wbase = wid * _EPW

    pltpu.sync_copy(zeros_hbm, table_sh.at[pl.ds(sid * _ROWS_PER_TILE,
                                                 _ROWS_PER_TILE)])
    pltpu.sync_copy(coef_hbm, coef_v)

    init_row = jnp.where(lax.iota(jnp.int32, 16) == 9,
                         jnp.float32(1.0), jnp.float32(0.0))

    def init_body(i, _):
        msg_v[i] = init_row
        return 0

    lax.fori_loop(0, _B, init_body, 0)

    plsc.subcore_barrier()

    lanes = lax.iota(jnp.int32, 16)

    def block_body(blk, _):
        base = wbase + blk * _B
        for j in range(_NCH):
            pltpu.sync_copy(ei_hbm.at[pl.ds(base + j * _CHUNK, _CHUNK)],
                            src_v.at[j])
            pltpu.sync_copy(
                ei_hbm.at[pl.ds(N_EDGES + base + j * _CHUNK, _CHUNK)],
                dst_v.at[j])
        pltpu.sync_copy(sh0_hbm.at[pl.ds(base, _B)], sh0_v)
        pltpu.sync_copy(sh1_hbm.at[pl.ds(base * 3, 3 * _B)], sh1_v)
        pltpu.sync_copy(sh2_hbm.at[pl.ds(base * 5, 5 * _B)], sh2_v)
        descs = [pltpu.async_copy(node_hbm.at[src_v.at[j]],
                                  rows_v.at[pl.ds(j * _CHUNK, _CHUNK)], sem)
                 for j in range(_NCH)]
        for d in descs:
            d.wait()

        def group_body(g, _):
            r = g * 16
            eidx = lanes + r
            shv = [sh0_v[pl.ds(r, 16)]]
            for i in range(3):
                shv.append(plsc.load_gather(sh1_v, [eidx * 3 + i]))
            for i in range(5):
                shv.append(plsc.load_gather(sh2_v, [eidx * 5 + i]))
            xv = [plsc.load_gather(
                rows_v, [eidx, jnp.full((16,), j, jnp.int32)])
                for j in range(TOTAL_DIM)]
            accs = _emit_bilinear(shv, xv, lambda cid_: coef_v[cid_])
            for k in range(TOTAL_DIM):
                plsc.store_scatter(
                    msg_v, [eidx, jnp.full((16,), k, jnp.int32)], accs[k])
            return 0

        lax.fori_loop(0, _GRP, group_body, 0)
        for j in range(_NCH):
            pltpu.sync_copy(msg_v.at[pl.ds(j * _CHUNK, _CHUNK)],
                            table_sh.at[dst_v.at[j]], add=True)
        return 0

    lax.fori_loop(0, _NB, block_body, 0)

    plsc.subcore_barrier()
    row0 = sid * _ROWS_PER_TILE
    pltpu.sync_copy(table_sh.at[pl.ds(row0, _ROWS_PER_TILE)],
                    out_hbm.at[cid, pl.ds(row0, _ROWS_PER_TILE)])


@jax.jit
def _sc_messages(node_pad, eflat, sh0, sh1, sh2, coef_tab, zeros):
    mesh = plsc.VectorSubcoreMesh(core_axis_name="c", subcore_axis_name="s")
    return pl.kernel(
        _sc_body,
        out_type=jax.ShapeDtypeStruct((_NC, _N_PAD, 16), jnp.float32),
        mesh=mesh,
        compiler_params=pltpu.CompilerParams(needs_layout_passes=False,
                                             use_tc_tiling_on_sc=False),
        scratch_types=[
            pltpu.VMEM((_NCH, _CHUNK), jnp.int32),
            pltpu.VMEM((_NCH, _CHUNK), jnp.int32),
            pltpu.VMEM((_B, 16), jnp.float32),
            pltpu.VMEM((_B,), jnp.float32),
            pltpu.VMEM((3 * _B,), jnp.float32),
            pltpu.VMEM((5 * _B,), jnp.float32),
            pltpu.VMEM((_B, 16), jnp.float32),
            pltpu.VMEM((_N_COEF_PAD, 16), jnp.float32),
            pltpu.VMEM_SHARED((_N_PAD, 16), jnp.float32),
            pltpu.SemaphoreType.DMA,
        ],
    )(node_pad, eflat, sh0, sh1, sh2, coef_tab, zeros)


_FR = 2000


def _fin_body(p_ref, o_ref):
    a = p_ref[0] + p_ref[1]
    cnt = jnp.maximum(a[:, 9:10], 1.0)
    o_ref[...] = a[:, :TOTAL_DIM] / cnt


@jax.jit
def _finalize(partials):
    return pl.pallas_call(
        _fin_body,
        grid=(N_NODES // _FR,),
        in_specs=[pl.BlockSpec((_NC, _FR, 16), lambda i: (0, i, 0))],
        out_specs=pl.BlockSpec((_FR, TOTAL_DIM), lambda i: (i, 0)),
        out_shape=jax.ShapeDtypeStruct((N_NODES, TOTAL_DIM), jnp.float32),
    )(partials)


def kernel(node_irreps, edge_index, sh_edge_0, sh_edge_1, sh_edge_2, w):
    node_pad = jnp.pad(node_irreps, ((0, 0), (0, 16 - TOTAL_DIM)))
    eflat = edge_index.reshape(2 * N_EDGES)
    sh0 = sh_edge_0.reshape(N_EDGES)
    sh1 = sh_edge_1.reshape(3 * N_EDGES)
    sh2 = sh_edge_2.reshape(5 * N_EDGES)
    coefs = w[jnp.asarray(_COEF_COMBO)] * jnp.asarray(_COEF_ABS)
    coefs = jnp.pad(coefs, (0, _N_COEF_PAD - _N_COEF))
    coef_tab = jnp.broadcast_to(coefs[:, None], (_N_COEF_PAD, 16))
    zeros = jnp.zeros((_ROWS_PER_TILE, 16), jnp.float32)
    partials = _sc_messages(node_pad, eflat, sh0, sh1, sh2,
                            jnp.asarray(coef_tab), zeros)
    return _finalize(partials)


# speedup vs baseline: 1.4382x; 1.0591x over previous
import math
from fractions import Fraction
import functools

import numpy as np
import jax
import jax.numpy as jnp
from jax import lax
from jax.experimental import pallas as pl
from jax.experimental.pallas import tpu as pltpu
from jax.experimental.pallas import tpu_sc as plsc

N_NODES = 50000
N_EDGES = 1600000
IRREP_DIMS = (1, 3, 5)
MAX_SH_DEGREE = 2
TOTAL_DIM = 9


def _su2_cg(j1m1, j2m2, j3m3):
    j1, m1 = j1m1
    j2, m2 = j2m2
    j3, m3 = j3m3
    if m3 != m1 + m2:
        return 0.0
    vmin = int(max(-j1 + j2 + m3, -j1 + m1, 0))
    vmax = int(min(j2 + j3 + m1, j3 - j1 + j2, j3 + m3))

    def f(n):
        return math.factorial(round(n))

    C = ((2.0 * j3 + 1.0) * Fraction(
        f(j3 + j1 - j2) * f(j3 - j1 + j2) * f(j1 + j2 - j3) * f(j3 + m3) * f(j3 - m3),
        f(j1 + j2 + j3 + 1) * f(j1 - m1) * f(j1 + m1) * f(j2 - m2) * f(j2 + m2))) ** 0.5
    S = 0
    for v in range(vmin, vmax + 1):
        S += (-1) ** int(v + j2 + m2) * Fraction(
            f(j2 + j3 + m1 - v) * f(j1 - m1 + v),
            f(v) * f(j3 - j1 + j2 - v) * f(j3 + m3 - v) * f(v + j1 - j2 - m3))
    return float(C * S)


def _su2_clebsch_gordan(j1, j2, j3):
    mat = np.zeros((2 * j1 + 1, 2 * j2 + 1, 2 * j3 + 1), dtype=np.float64)
    for m1 in range(-j1, j1 + 1):
        for m2 in range(-j2, j2 + 1):
            if abs(m1 + m2) <= j3:
                mat[j1 + m1, j2 + m2, j3 + m1 + m2] = _su2_cg(
                    (j1, m1), (j2, m2), (j3, m1 + m2))
    return mat


def _change_basis_real_to_complex(l):
    q = np.zeros((2 * l + 1, 2 * l + 1), dtype=np.complex128)
    for m in range(-l, 0):
        q[l + m, l + abs(m)] = 1 / 2 ** 0.5
        q[l + m, l - abs(m)] = -1j / 2 ** 0.5
    q[l, l] = 1
    for m in range(1, l + 1):
        q[l + m, l + abs(m)] = (-1) ** m / 2 ** 0.5
        q[l + m, l - abs(m)] = 1j * (-1) ** m / 2 ** 0.5
    return (-1j) ** l * q


def _so3_clebsch_gordan(l1, l2, l3):
    Q1 = _change_basis_real_to_complex(l1)
    Q2 = _change_basis_real_to_complex(l2)
    Q3 = _change_basis_real_to_complex(l3)
    C = _su2_clebsch_gordan(l1, l2, l3).astype(np.complex128)
    C = np.einsum('ij,kl,mn,ikn->jlm', Q1, Q2, np.conj(Q3.T), C)
    return np.real(C).astype(np.float32)


def _build_cg_combos():
    ls = [(d - 1) // 2 for d in IRREP_DIMS]
    combos = []
    for e in range(MAX_SH_DEGREE + 1):
        for ni in range(len(IRREP_DIMS)):
            for oi in range(len(IRREP_DIMS)):
                nl, ol = ls[ni], ls[oi]
                if not (abs(e - nl) <= ol <= e + nl):
                    continue
                cg = _so3_clebsch_gordan(e, nl, ol)
                if np.abs(cg).sum() > 1e-10:
                    combos.append((e, ni, oi, cg))
    return combos


def _build_tables():
    combos = _build_cg_combos()
    offs = [0, 1, 4, 9]
    terms = []
    for idx, (e, ni, oi, cg) in enumerate(combos):
        I, J, K = cg.shape
        for i in range(I):
            for j in range(J):
                for k in range(K):
                    v = float(cg[i, j, k])
                    if abs(v) > 1e-8:
                        terms.append((offs[e] + i, offs[ni] + j, offs[oi] + k, idx, v))
    coef_ids = {}
    coef_rows = []
    groups = [{} for _ in range(TOTAL_DIM)]
    for (i, j, k, c, v) in terms:
        key = (c, round(abs(v), 10))
        if key not in coef_ids:
            coef_ids[key] = len(coef_rows)
            coef_rows.append(key)
        cid = coef_ids[key]
        groups[k].setdefault(cid, []).append((i, j, 1 if v > 0 else -1))
    combo_idx = np.asarray([c for (c, av) in coef_rows], dtype=np.int32)
    absval = np.asarray([av for (c, av) in coef_rows], dtype=np.float32)
    groups = [list(g.items()) for g in groups]
    return combo_idx, absval, groups


_COEF_COMBO, _COEF_ABS, _GROUPS = _build_tables()
_N_COEF = _COEF_COMBO.shape[0]
_N_COEF_PAD = ((_N_COEF + 7) // 8) * 8


def _emit_bilinear(shv, xv, coef_of):
    prod_cache = {}

    def prod(i, j):
        if (i, j) not in prod_cache:
            prod_cache[(i, j)] = shv[i] * xv[j]
        return prod_cache[(i, j)]

    accs = []
    for k in range(TOTAL_DIM):
        acc = None
        for cid, tlist in _GROUPS[k]:
            tl = sorted(tlist, key=lambda t: -t[2])
            neg = tl[0][2] < 0
            if neg:
                tl = [(i, j, -s) for (i, j, s) in tl]
            br = None
            for (i, j, s) in tl:
                p = prod(i, j)
                if br is None:
                    br = p
                elif s > 0:
                    br = br + p
                else:
                    br = br - p
            v = coef_of(cid) * br
            if acc is None:
                acc = (-v) if neg else v
            elif neg:
                acc = acc - v
            else:
                acc = acc + v
        accs.append(acc)
    return accs


_NC = 2
_NS = 16
_NW = _NC * _NS
_EPW = N_EDGES // _NW
_B = 400
_NB = _EPW // _B
_GRP = _B // 16
_CHUNK = 80
_NCH = _B // _CHUNK
_ROWS_PER_TILE = 3128
_N_PAD = _ROWS_PER_TILE * _NS


def _sc_body(node_hbm, src_hbm, dst_hbm, sh0_hbm, sh1_hbm, sh2_hbm,
             coef_hbm, zeros_hbm, out_hbm,
             src_st, dst_st, src_v, dst_v, rows_v, sh0_v, sh1_v, sh2_v,
             msg_v, coef_v, table_sh, sem):
    cid = lax.axis_index("c")
    sid = lax.axis_index("s")
    wid = sid * _NC + cid
    wbase = wid * _EPW

    pltpu.sync_copy(zeros_hbm, table_sh.at[pl.ds(sid * _ROWS_PER_TILE,
                                                 _ROWS_PER_TILE)])
    pltpu.sync_copy(coef_hbm, coef_v)

    init_row = jnp.where(lax.iota(jnp.int32, 16) == 9,
                         jnp.float32(1.0), jnp.float32(0.0))

    def init_body(i, _):
        msg_v[i] = init_row
        return 0

    lax.fori_loop(0, _B, init_body, 0)

    plsc.subcore_barrier()

    lanes = lax.iota(jnp.int32, 16)

    def block_body(blk, _):
        base = wbase + blk * _B
        pltpu.sync_copy(src_hbm.at[pl.ds(base, _B)], src_st)
        pltpu.sync_copy(dst_hbm.at[pl.ds(base, _B)], dst_st)
        for j in range(_NCH):
            for t in range(_CHUNK // 16):
                src_v[j, pl.ds(t * 16, 16)] = (
                    src_st[pl.ds(j * _CHUNK + t * 16, 16)])
                dst_v[j, pl.ds(t * 16, 16)] = (
                    dst_st[pl.ds(j * _CHUNK + t * 16, 16)])
        pltpu.sync_copy(sh0_hbm.at[pl.ds(base, _B)], sh0_v)
        pltpu.sync_copy(sh1_hbm.at[pl.ds(base * 3, 3 * _B)], sh1_v)
        pltpu.sync_copy(sh2_hbm.at[pl.ds(base * 5, 5 * _B)], sh2_v)
        descs = [pltpu.async_copy(node_hbm.at[src_v.at[j]],
                                  rows_v.at[pl.ds(j * _CHUNK, _CHUNK)], sem)
                 for j in range(_NCH)]
        for d in descs:
            d.wait()

        def group_body(g, _):
            r = g * 16
            eidx = lanes + r
            shv = [sh0_v[pl.ds(r, 16)]]
            for i in range(3):
                shv.append(plsc.load_gather(sh1_v, [eidx * 3 + i]))
            for i in range(5):
                shv.append(plsc.load_gather(sh2_v, [eidx * 5 + i]))
            xv = [plsc.load_gather(
                rows_v, [eidx, jnp.full((16,), j, jnp.int32)])
                for j in range(TOTAL_DIM)]
            accs = _emit_bilinear(shv, xv, lambda cid_: coef_v[cid_])
            for k in range(TOTAL_DIM):
                plsc.store_scatter(
                    msg_v, [eidx, jnp.full((16,), k, jnp.int32)], accs[k])
            return 0

        lax.fori_loop(0, _GRP, group_body, 0)
        for j in range(_NCH):
            pltpu.sync_copy(msg_v.at[pl.ds(j * _CHUNK, _CHUNK)],
                            table_sh.at[dst_v.at[j]], add=True)
        return 0

    lax.fori_loop(0, _NB, block_body, 0)

    plsc.subcore_barrier()
    row0 = sid * _ROWS_PER_TILE
    pltpu.sync_copy(table_sh.at[pl.ds(row0, _ROWS_PER_TILE)],
                    out_hbm.at[cid, pl.ds(row0, _ROWS_PER_TILE)])


@jax.jit
def _sc_messages(node_pad, src, dst, sh0, sh1, sh2, coef_tab, zeros):
    mesh = plsc.VectorSubcoreMesh(core_axis_name="c", subcore_axis_name="s")
    return pl.kernel(
        _sc_body,
        out_type=jax.ShapeDtypeStruct((_NC, _N_PAD, 16), jnp.float32),
        mesh=mesh,
        compiler_params=pltpu.CompilerParams(needs_layout_passes=False,
                                             use_tc_tiling_on_sc=False),
        scratch_types=[
            pltpu.VMEM((_B,), jnp.int32),
            pltpu.VMEM((_B,), jnp.int32),
            pltpu.VMEM((_NCH, _CHUNK), jnp.int32),
            pltpu.VMEM((_NCH, _CHUNK), jnp.int32),
            pltpu.VMEM((_B, 16), jnp.float32),
            pltpu.VMEM((_B,), jnp.float32),
            pltpu.VMEM((3 * _B,), jnp.float32),
            pltpu.VMEM((5 * _B,), jnp.float32),
            pltpu.VMEM((_B, 16), jnp.float32),
            pltpu.VMEM((_N_COEF_PAD, 16), jnp.float32),
            pltpu.VMEM_SHARED((_N_PAD, 16), jnp.float32),
            pltpu.SemaphoreType.DMA,
        ],
    )(node_pad, src, dst, sh0, sh1, sh2, coef_tab, zeros)


_EB = 131072


def _split_body(ei_ref, s_ref, d_ref):
    s_ref[...] = ei_ref[0]
    d_ref[...] = ei_ref[1]


@jax.jit
def _split_edges(edge_index):
    return pl.pallas_call(
        _split_body,
        grid=(pl.cdiv(N_EDGES, _EB),),
        in_specs=[pl.BlockSpec((2, _EB), lambda i: (0, i))],
        out_specs=[pl.BlockSpec((_EB,), lambda i: (i,)),
                   pl.BlockSpec((_EB,), lambda i: (i,))],
        out_shape=[jax.ShapeDtypeStruct((N_EDGES,), jnp.int32),
                   jax.ShapeDtypeStruct((N_EDGES,), jnp.int32)],
    )(edge_index)


_FR = 2000


def _fin_body(p_ref, o_ref):
    a = p_ref[0] + p_ref[1]
    cnt = jnp.maximum(a[:, 9:10], 1.0)
    o_ref[...] = a[:, :TOTAL_DIM] / cnt


@jax.jit
def _finalize(partials):
    return pl.pallas_call(
        _fin_body,
        grid=(N_NODES // _FR,),
        in_specs=[pl.BlockSpec((_NC, _FR, 16), lambda i: (0, i, 0))],
        out_specs=pl.BlockSpec((_FR, TOTAL_DIM), lambda i: (i, 0)),
        out_shape=jax.ShapeDtypeStruct((N_NODES, TOTAL_DIM), jnp.float32),
    )(partials)


def kernel(node_irreps, edge_index, sh_edge_0, sh_edge_1, sh_edge_2, w):
    node_pad = jnp.pad(node_irreps, ((0, 0), (0, 16 - TOTAL_DIM)))
    src, dst = _split_edges(edge_index)
    sh0 = sh_edge_0.reshape(N_EDGES)
    sh1 = sh_edge_1.reshape(3 * N_EDGES)
    sh2 = sh_edge_2.reshape(5 * N_EDGES)
    coefs = w[jnp.asarray(_COEF_COMBO)] * jnp.asarray(_COEF_ABS)
    coefs = jnp.pad(coefs, (0, _N_COEF_PAD - _N_COEF))
    coef_tab = jnp.broadcast_to(coefs[:, None], (_N_COEF_PAD, 16))
    zeros = jnp.zeros((_ROWS_PER_TILE, 16), jnp.float32)
    partials = _sc_messages(node_pad, src, dst, sh0, sh1, sh2,
                            jnp.asarray(coef_tab), zeros)
    return _finalize(partials)


# speedup vs baseline: 4.7905x; 3.3308x over previous
import math
from fractions import Fraction
import functools

import numpy as np
import jax
import jax.numpy as jnp
from jax import lax
from jax.experimental import pallas as pl
from jax.experimental.pallas import tpu as pltpu
from jax.experimental.pallas import tpu_sc as plsc

N_NODES = 50000
N_EDGES = 1600000
IRREP_DIMS = (1, 3, 5)
MAX_SH_DEGREE = 2
TOTAL_DIM = 9


def _su2_cg(j1m1, j2m2, j3m3):
    j1, m1 = j1m1
    j2, m2 = j2m2
    j3, m3 = j3m3
    if m3 != m1 + m2:
        return 0.0
    vmin = int(max(-j1 + j2 + m3, -j1 + m1, 0))
    vmax = int(min(j2 + j3 + m1, j3 - j1 + j2, j3 + m3))

    def f(n):
        return math.factorial(round(n))

    C = ((2.0 * j3 + 1.0) * Fraction(
        f(j3 + j1 - j2) * f(j3 - j1 + j2) * f(j1 + j2 - j3) * f(j3 + m3) * f(j3 - m3),
        f(j1 + j2 + j3 + 1) * f(j1 - m1) * f(j1 + m1) * f(j2 - m2) * f(j2 + m2))) ** 0.5
    S = 0
    for v in range(vmin, vmax + 1):
        S += (-1) ** int(v + j2 + m2) * Fraction(
            f(j2 + j3 + m1 - v) * f(j1 - m1 + v),
            f(v) * f(j3 - j1 + j2 - v) * f(j3 + m3 - v) * f(v + j1 - j2 - m3))
    return float(C * S)


def _su2_clebsch_gordan(j1, j2, j3):
    mat = np.zeros((2 * j1 + 1, 2 * j2 + 1, 2 * j3 + 1), dtype=np.float64)
    for m1 in range(-j1, j1 + 1):
        for m2 in range(-j2, j2 + 1):
            if abs(m1 + m2) <= j3:
                mat[j1 + m1, j2 + m2, j3 + m1 + m2] = _su2_cg(
                    (j1, m1), (j2, m2), (j3, m1 + m2))
    return mat


def _change_basis_real_to_complex(l):
    q = np.zeros((2 * l + 1, 2 * l + 1), dtype=np.complex128)
    for m in range(-l, 0):
        q[l + m, l + abs(m)] = 1 / 2 ** 0.5
        q[l + m, l - abs(m)] = -1j / 2 ** 0.5
    q[l, l] = 1
    for m in range(1, l + 1):
        q[l + m, l + abs(m)] = (-1) ** m / 2 ** 0.5
        q[l + m, l - abs(m)] = 1j * (-1) ** m / 2 ** 0.5
    return (-1j) ** l * q


def _so3_clebsch_gordan(l1, l2, l3):
    Q1 = _change_basis_real_to_complex(l1)
    Q2 = _change_basis_real_to_complex(l2)
    Q3 = _change_basis_real_to_complex(l3)
    C = _su2_clebsch_gordan(l1, l2, l3).astype(np.complex128)
    C = np.einsum('ij,kl,mn,ikn->jlm', Q1, Q2, np.conj(Q3.T), C)
    return np.real(C).astype(np.float32)


def _build_cg_combos():
    ls = [(d - 1) // 2 for d in IRREP_DIMS]
    combos = []
    for e in range(MAX_SH_DEGREE + 1):
        for ni in range(len(IRREP_DIMS)):
            for oi in range(len(IRREP_DIMS)):
                nl, ol = ls[ni], ls[oi]
                if not (abs(e - nl) <= ol <= e + nl):
                    continue
                cg = _so3_clebsch_gordan(e, nl, ol)
                if np.abs(cg).sum() > 1e-10:
                    combos.append((e, ni, oi, cg))
    return combos


def _build_tables():
    combos = _build_cg_combos()
    offs = [0, 1, 4, 9]
    terms = []
    for idx, (e, ni, oi, cg) in enumerate(combos):
        I, J, K = cg.shape
        for i in range(I):
            for j in range(J):
                for k in range(K):
                    v = float(cg[i, j, k])
                    if abs(v) > 1e-8:
                        terms.append((offs[e] + i, offs[ni] + j, offs[oi] + k, idx, v))
    coef_ids = {}
    coef_rows = []
    groups = [{} for _ in range(TOTAL_DIM)]
    for (i, j, k, c, v) in terms:
        key = (c, round(abs(v), 10))
        if key not in coef_ids:
            coef_ids[key] = len(coef_rows)
            coef_rows.append(key)
        cid = coef_ids[key]
        groups[k].setdefault(cid, []).append((i, j, 1 if v > 0 else -1))
    combo_idx = np.asarray([c for (c, av) in coef_rows], dtype=np.int32)
    absval = np.asarray([av for (c, av) in coef_rows], dtype=np.float32)
    groups = [list(g.items()) for g in groups]
    return combo_idx, absval, groups


_COEF_COMBO, _COEF_ABS, _GROUPS = _build_tables()
_N_COEF = _COEF_COMBO.shape[0]
_N_COEF_PAD = ((_N_COEF + 7) // 8) * 8


def _emit_bilinear(shv, xv, coef_of):
    prod_cache = {}

    def prod(i, j):
        if (i, j) not in prod_cache:
            prod_cache[(i, j)] = shv[i] * xv[j]
        return prod_cache[(i, j)]

    accs = []
    for k in range(TOTAL_DIM):
        acc = None
        for cid, tlist in _GROUPS[k]:
            tl = sorted(tlist, key=lambda t: -t[2])
            neg = tl[0][2] < 0
            if neg:
                tl = [(i, j, -s) for (i, j, s) in tl]
            br = None
            for (i, j, s) in tl:
                p = prod(i, j)
                if br is None:
                    br = p
                elif s > 0:
                    br = br + p
                else:
                    br = br - p
            v = coef_of(cid) * br
            if acc is None:
                acc = (-v) if neg else v
            elif neg:
                acc = acc - v
            else:
                acc = acc + v
        accs.append(acc)
    return accs


_NC = 2
_NS = 16
_NW = _NC * _NS
_EPW = N_EDGES // _NW
_B = 400
_NB = _EPW // _B
_GRP = _B // 16
_CHUNK = 80
_NCH = _B // _CHUNK
_ROWS_PER_TILE = 3128
_N_PAD = _ROWS_PER_TILE * _NS


def _sc_body(node_hbm, src_hbm, dst_hbm, sh0_hbm, sh1_hbm, sh2_hbm,
             coef_hbm, zeros_hbm, out_hbm,
             src_st, dst_st, src_v, dst_v, rows_v, sh0_v, sh1_v, sh2_v,
             msg_v, coef_v, table_sh, sem):
    cid = lax.axis_index("c")
    sid = lax.axis_index("s")
    wid = sid * _NC + cid
    wbase = wid * _EPW

    pltpu.sync_copy(zeros_hbm, table_sh.at[pl.ds(sid * _ROWS_PER_TILE,
                                                 _ROWS_PER_TILE)])
    pltpu.sync_copy(coef_hbm, coef_v)

    init_row = jnp.where(lax.iota(jnp.int32, 16) == 9,
                         jnp.float32(1.0), jnp.float32(0.0))

    def init_body(i, _):
        msg_v[i] = init_row
        return 0

    lax.fori_loop(0, _B, init_body, 0)

    plsc.subcore_barrier()

    lanes = lax.iota(jnp.int32, 16)

    def block_body(blk, _):
        base = wbase + blk * _B
        pltpu.sync_copy(src_hbm.at[pl.ds(base, _B)], src_st)
        pltpu.sync_copy(dst_hbm.at[pl.ds(base, _B)], dst_st)
        for j in range(_NCH):
            for t in range(_CHUNK // 16):
                src_v[j, pl.ds(t * 16, 16)] = (
                    src_st[pl.ds(j * _CHUNK + t * 16, 16)])
                dst_v[j, pl.ds(t * 16, 16)] = (
                    dst_st[pl.ds(j * _CHUNK + t * 16, 16)])
        pltpu.sync_copy(sh0_hbm.at[pl.ds(base, _B)], sh0_v)
        pltpu.sync_copy(sh1_hbm.at[:, pl.ds(base, _B)], sh1_v)
        pltpu.sync_copy(sh2_hbm.at[:, pl.ds(base, _B)], sh2_v)
        descs = [pltpu.async_copy(node_hbm.at[src_v.at[j]],
                                  rows_v.at[pl.ds(j * _CHUNK, _CHUNK)], sem)
                 for j in range(_NCH)]
        for d in descs:
            d.wait()

        def group_body(g, _):
            r = g * 16
            eidx = lanes + r
            shv = [sh0_v[pl.ds(r, 16)]]
            for i in range(3):
                shv.append(sh1_v[i, pl.ds(r, 16)])
            for i in range(5):
                shv.append(sh2_v[i, pl.ds(r, 16)])
            xv = [plsc.load_gather(
                rows_v, [eidx, jnp.full((16,), j, jnp.int32)])
                for j in range(TOTAL_DIM)]
            accs = _emit_bilinear(shv, xv, lambda cid_: coef_v[cid_])
            for k in range(TOTAL_DIM):
                plsc.store_scatter(
                    msg_v, [eidx, jnp.full((16,), k, jnp.int32)], accs[k])
            return 0

        lax.fori_loop(0, _GRP, group_body, 0)
        for j in range(_NCH):
            pltpu.sync_copy(msg_v.at[pl.ds(j * _CHUNK, _CHUNK)],
                            table_sh.at[dst_v.at[j]], add=True)
        return 0

    lax.fori_loop(0, _NB, block_body, 0)

    plsc.subcore_barrier()
    row0 = sid * _ROWS_PER_TILE
    pltpu.sync_copy(table_sh.at[pl.ds(row0, _ROWS_PER_TILE)],
                    out_hbm.at[cid, pl.ds(row0, _ROWS_PER_TILE)])


@jax.jit
def _sc_messages(node_pad, src, dst, sh0, sh1, sh2, coef_tab, zeros):
    mesh = plsc.VectorSubcoreMesh(core_axis_name="c", subcore_axis_name="s")
    return pl.kernel(
        _sc_body,
        out_type=jax.ShapeDtypeStruct((_NC, _N_PAD, 16), jnp.float32),
        mesh=mesh,
        compiler_params=pltpu.CompilerParams(needs_layout_passes=False,
                                             use_tc_tiling_on_sc=False),
        scratch_types=[
            pltpu.VMEM((_B,), jnp.int32),
            pltpu.VMEM((_B,), jnp.int32),
            pltpu.VMEM((_NCH, _CHUNK), jnp.int32),
            pltpu.VMEM((_NCH, _CHUNK), jnp.int32),
            pltpu.VMEM((_B, 16), jnp.float32),
            pltpu.VMEM((_B,), jnp.float32),
            pltpu.VMEM((3, _B), jnp.float32),
            pltpu.VMEM((5, _B), jnp.float32),
            pltpu.VMEM((_B, 16), jnp.float32),
            pltpu.VMEM((_N_COEF_PAD, 16), jnp.float32),
            pltpu.VMEM_SHARED((_N_PAD, 16), jnp.float32),
            pltpu.SemaphoreType.DMA,
        ],
    )(node_pad, src, dst, sh0, sh1, sh2, coef_tab, zeros)


_EB = 131072


def _split_body(ei_ref, s_ref, d_ref):
    s_ref[...] = ei_ref[0]
    d_ref[...] = ei_ref[1]


@jax.jit
def _split_edges(edge_index):
    return pl.pallas_call(
        _split_body,
        grid=(pl.cdiv(N_EDGES, _EB),),
        in_specs=[pl.BlockSpec((2, _EB), lambda i: (0, i))],
        out_specs=[pl.BlockSpec((_EB,), lambda i: (i,)),
                   pl.BlockSpec((_EB,), lambda i: (i,))],
        out_shape=[jax.ShapeDtypeStruct((N_EDGES,), jnp.int32),
                   jax.ShapeDtypeStruct((N_EDGES,), jnp.int32)],
    )(edge_index)


_FR = 2000


def _fin_body(p_ref, o_ref):
    a = p_ref[0] + p_ref[1]
    cnt = jnp.maximum(a[:, 9:10], 1.0)
    o_ref[...] = a[:, :TOTAL_DIM] / cnt


@jax.jit
def _finalize(partials):
    return pl.pallas_call(
        _fin_body,
        grid=(N_NODES // _FR,),
        in_specs=[pl.BlockSpec((_NC, _FR, 16), lambda i: (0, i, 0))],
        out_specs=pl.BlockSpec((_FR, TOTAL_DIM), lambda i: (i, 0)),
        out_shape=jax.ShapeDtypeStruct((N_NODES, TOTAL_DIM), jnp.float32),
    )(partials)


def kernel(node_irreps, edge_index, sh_edge_0, sh_edge_1, sh_edge_2, w):
    node_pad = jnp.pad(node_irreps, ((0, 0), (0, 16 - TOTAL_DIM)))
    src, dst = _split_edges(edge_index)
    sh0 = sh_edge_0.reshape(N_EDGES)
    sh1 = sh_edge_1.T
    sh2 = sh_edge_2.T
    coefs = w[jnp.asarray(_COEF_COMBO)] * jnp.asarray(_COEF_ABS)
    coefs = jnp.pad(coefs, (0, _N_COEF_PAD - _N_COEF))
    coef_tab = jnp.broadcast_to(coefs[:, None], (_N_COEF_PAD, 16))
    zeros = jnp.zeros((_ROWS_PER_TILE, 16), jnp.float32)
    partials = _sc_messages(node_pad, src, dst, sh0, sh1, sh2,
                            jnp.asarray(coef_tab), zeros)
    return _finalize(partials)
